# Initial kernel scaffold; baseline (speedup 1.0000x reference)
import jax
import jax.numpy as jnp
from jax.experimental import pallas as pl


def kernel(feature, feature_lens, input_ids, attention_mask, labels, embed_table, prompt_embedding, W_proj, b_proj):
    raise NotImplementedError("write your pallas kernel here")


# speedup vs baseline: 1.0232x; 1.0232x over previous
import jax
import jax.numpy as jnp
from jax.experimental import pallas as pl
from jax.experimental.pallas import tpu as pltpu

_SPEECH_ID = 31999
_ST_ID = 31995
_ET_ID = 31996
_SS_ID = 31997
_ES_ID = 31998
_IGNORE = -100

_R = 8


def _proj_body(f_ref, w_ref, b_ref, o_ref):
    o_ref[...] = (
        jnp.dot(f_ref[0], w_ref[...], preferred_element_type=jnp.float32)
        + b_ref[...]
    )[None]


def _merge_body(tok_ref, si_ref, code_ref, *refs):
    emb_refs = refs[:_R]
    sp_ref, prompt_ref, out_ref = refs[_R], refs[_R + 1], refs[_R + 2]
    b = pl.program_id(0)
    jt = pl.program_id(1)
    for r in range(_R):
        j = jt * _R + r
        c = code_ref[b, j]
        si = si_ref[b, j]
        emb_row = emb_refs[r][...].reshape(8, 128)
        sp_row = sp_ref[pl.ds(0, 1), pl.ds(si, 1), :, :].reshape(8, 128)
        pr_row = prompt_ref[pl.ds(jnp.clip(c, 0, 3), 1), :, :].reshape(8, 128)
        row = jnp.where(
            c == 4,
            emb_row,
            jnp.where(c == 5, sp_row, jnp.where(c >= 0, pr_row, 0.0)),
        )
        out_ref[pl.ds(0, 1), pl.ds(r, 1), :, :] = row.reshape(1, 1, 8, 128)


@jax.jit
def kernel(feature, feature_lens, input_ids, attention_mask, labels,
           embed_table, prompt_embedding, W_proj, b_proj):
    B, S, d_enc = feature.shape
    L = input_ids.shape[1]
    V, d = embed_table.shape
    L_out = L - 1 + S
    L_pad = ((L_out + _R - 1) // _R) * _R
    NT = L_pad // _R
    dt = d // 128

    speech_feats = pl.pallas_call(
        _proj_body,
        grid=(B,),
        in_specs=[
            pl.BlockSpec((1, S, d_enc), lambda b: (b, 0, 0)),
            pl.BlockSpec((d_enc, d), lambda b: (0, 0)),
            pl.BlockSpec((1, d), lambda b: (0, 0)),
        ],
        out_specs=pl.BlockSpec((1, S, d), lambda b: (b, 0, 0)),
        out_shape=jax.ShapeDtypeStruct((B, S, d), jnp.float32),
    )(feature, W_proj, b_proj.reshape(1, d))

    T = attention_mask.astype(jnp.int32).sum(axis=1)
    pos = jnp.argmax(input_ids == _SPEECH_ID, axis=1).astype(jnp.int32)
    sl = feature_lens.astype(jnp.int32)
    total = T - 1 + sl
    offset = L_out - total
    j = jnp.arange(L_pad, dtype=jnp.int32)[None, :]
    k = j - offset[:, None]
    is_sp = (k >= pos[:, None]) & (k < (pos + sl)[:, None])
    text_idx = jnp.where(k < pos[:, None], k, k - sl[:, None] + 1)
    text_idx = jnp.clip(text_idx, 0, L - 1)
    si = jnp.clip(k - pos[:, None], 0, S - 1).astype(jnp.int32)
    valid = (k >= 0) & (k < total[:, None]) & (j < L_out)
    tok_raw = jnp.take_along_axis(input_ids, text_idx, axis=1)
    widx = jnp.full_like(tok_raw, -1)
    for t, w in ((_ST_ID, 0), (_ET_ID, 1), (_SS_ID, 2), (_ES_ID, 3)):
        widx = jnp.where(tok_raw == t, w, widx)
    code = jnp.where(
        ~valid, -1, jnp.where(is_sp, 5, jnp.where(widx >= 0, widx, 4))
    ).astype(jnp.int32)
    tok = jnp.clip(jnp.where(code == 4, tok_raw, 0), 0, V - 1).astype(jnp.int32)

    prompt_pad = jnp.zeros((8, d), jnp.float32).at[:4].set(prompt_embedding)

    emb_t = embed_table.reshape(V, dt, 128)
    sp_t = speech_feats.reshape(B, S, dt, 128)
    prompt_t = prompt_pad.reshape(8, dt, 128)

    def emb_spec(r):
        def imap(b, jt, tok_ref, si_ref, code_ref):
            return (tok_ref[b, jt * _R + r], 0, 0)
        return pl.BlockSpec((1, dt, 128), imap)

    grid_spec = pltpu.PrefetchScalarGridSpec(
        num_scalar_prefetch=3,
        grid=(B, NT),
        in_specs=(
            [emb_spec(r) for r in range(_R)]
            + [
                pl.BlockSpec((1, S, dt, 128), lambda b, jt, *_: (b, 0, 0, 0)),
                pl.BlockSpec((8, dt, 128), lambda b, jt, *_: (0, 0, 0)),
            ]
        ),
        out_specs=pl.BlockSpec((1, _R, dt, 128), lambda b, jt, *_: (b, jt, 0, 0)),
    )
    merged = pl.pallas_call(
        _merge_body,
        grid_spec=grid_spec,
        out_shape=jax.ShapeDtypeStruct((B, L_pad, dt, 128), jnp.float32),
    )(tok, si, code, *([emb_t] * _R), sp_t, prompt_t)
    merged = merged.reshape(B, L_pad, d)[:, :L_out]

    lab = jnp.take_along_axis(labels, text_idx, axis=1)
    lab = jnp.where(is_sp, _IGNORE, lab)
    lab = jnp.where(valid, lab, _IGNORE).astype(jnp.int32)
    out_mask = valid[:, :L_out]
    out_labels = lab[:, :L_out]
    return merged, out_mask, out_labels


# speedup vs baseline: 1.3704x; 1.3393x over previous
import jax
import jax.numpy as jnp
from jax.experimental import pallas as pl
from jax.experimental.pallas import tpu as pltpu

_SPEECH_ID = 31999
_ST_ID = 31995
_ET_ID = 31996
_SS_ID = 31997
_ES_ID = 31998
_IGNORE = -100

_R = 16


def _proj_body(f_ref, w_ref, b_ref, o_ref):
    o_ref[...] = (
        jnp.dot(f_ref[0], w_ref[...], preferred_element_type=jnp.float32)
        + b_ref[...]
    )[None]


def _merge_body(tok_ref, si_ref, code_ref, *refs):
    emb_refs = refs[:_R]
    sp_ref, prompt_ref, out_ref = refs[_R], refs[_R + 1], refs[_R + 2]
    b = pl.program_id(0)
    jt = pl.program_id(1)
    for r in range(_R):
        j = jt * _R + r
        c = code_ref[b, j]
        si = si_ref[b, j]
        emb_row = emb_refs[r][...].reshape(8, 128)
        sp_row = sp_ref[pl.ds(0, 1), pl.ds(si, 1), :, :].reshape(8, 128)
        pr_row = prompt_ref[pl.ds(jnp.clip(c, 0, 3), 1), :, :].reshape(8, 128)
        row = jnp.where(
            c == 4,
            emb_row,
            jnp.where(c == 5, sp_row, jnp.where(c >= 0, pr_row, 0.0)),
        )
        out_ref[pl.ds(0, 1), pl.ds(r, 1), :, :] = row.reshape(1, 1, 8, 128)


@jax.jit
def kernel(feature, feature_lens, input_ids, attention_mask, labels,
           embed_table, prompt_embedding, W_proj, b_proj):
    B, S, d_enc = feature.shape
    L = input_ids.shape[1]
    V, d = embed_table.shape
    L_out = L - 1 + S
    L_pad = ((L_out + _R - 1) // _R) * _R
    NT = L_pad // _R
    dt = d // 128

    speech_feats = pl.pallas_call(
        _proj_body,
        grid=(B,),
        in_specs=[
            pl.BlockSpec((1, S, d_enc), lambda b: (b, 0, 0)),
            pl.BlockSpec((d_enc, d), lambda b: (0, 0)),
            pl.BlockSpec((1, d), lambda b: (0, 0)),
        ],
        out_specs=pl.BlockSpec((1, S, d), lambda b: (b, 0, 0)),
        out_shape=jax.ShapeDtypeStruct((B, S, d), jnp.float32),
    )(feature, W_proj, b_proj.reshape(1, d))

    T = attention_mask.astype(jnp.int32).sum(axis=1)
    pos = jnp.argmax(input_ids == _SPEECH_ID, axis=1).astype(jnp.int32)
    sl = feature_lens.astype(jnp.int32)
    total = T - 1 + sl
    offset = L_out - total
    j = jnp.arange(L_pad, dtype=jnp.int32)[None, :]
    k = j - offset[:, None]
    is_sp = (k >= pos[:, None]) & (k < (pos + sl)[:, None])
    text_idx = jnp.where(k < pos[:, None], k, k - sl[:, None] + 1)
    text_idx = jnp.clip(text_idx, 0, L - 1)
    si = jnp.clip(k - pos[:, None], 0, S - 1).astype(jnp.int32)
    valid = (k >= 0) & (k < total[:, None]) & (j < L_out)
    tok_raw = jnp.take_along_axis(input_ids, text_idx, axis=1)
    widx = jnp.full_like(tok_raw, -1)
    for t, w in ((_ST_ID, 0), (_ET_ID, 1), (_SS_ID, 2), (_ES_ID, 3)):
        widx = jnp.where(tok_raw == t, w, widx)
    code = jnp.where(
        ~valid, -1, jnp.where(is_sp, 5, jnp.where(widx >= 0, widx, 4))
    ).astype(jnp.int32)
    tok = jnp.clip(jnp.where(code == 4, tok_raw, 0), 0, V - 1).astype(jnp.int32)

    prompt_pad = jnp.zeros((8, d), jnp.float32).at[:4].set(prompt_embedding)

    emb_t = embed_table.reshape(V, dt, 128)
    sp_t = speech_feats.reshape(B, S, dt, 128)
    prompt_t = prompt_pad.reshape(8, dt, 128)

    def emb_spec(r):
        def imap(b, jt, tok_ref, si_ref, code_ref):
            return (tok_ref[b, jt * _R + r], 0, 0)
        return pl.BlockSpec((1, dt, 128), imap)

    grid_spec = pltpu.PrefetchScalarGridSpec(
        num_scalar_prefetch=3,
        grid=(B, NT),
        in_specs=(
            [emb_spec(r) for r in range(_R)]
            + [
                pl.BlockSpec((1, S, dt, 128), lambda b, jt, *_: (b, 0, 0, 0)),
                pl.BlockSpec((8, dt, 128), lambda b, jt, *_: (0, 0, 0)),
            ]
        ),
        out_specs=pl.BlockSpec((1, _R, dt, 128), lambda b, jt, *_: (b, jt, 0, 0)),
    )
    merged = pl.pallas_call(
        _merge_body,
        grid_spec=grid_spec,
        out_shape=jax.ShapeDtypeStruct((B, L_pad, dt, 128), jnp.float32),
    )(tok, si, code, *([emb_t] * _R), sp_t, prompt_t)
    merged = merged.reshape(B, L_pad, d)[:, :L_out]

    lab = jnp.take_along_axis(labels, text_idx, axis=1)
    lab = jnp.where(is_sp, _IGNORE, lab)
    lab = jnp.where(valid, lab, _IGNORE).astype(jnp.int32)
    out_mask = valid[:, :L_out]
    out_labels = lab[:, :L_out]
    return merged, out_mask, out_labels
